# Initial kernel scaffold; baseline (speedup 1.0000x reference)
import jax
import jax.numpy as jnp
from jax.experimental import pallas as pl


def kernel(fMRI_v, fMRI_a, params):
    raise NotImplementedError("write your pallas kernel here")


# speedup vs baseline: 12.4491x; 12.4491x over previous
import functools

import jax
import jax.numpy as jnp
from jax.experimental import pallas as pl
from jax.experimental.pallas import tpu as pltpu

B, T, N, C, NC = 4, 32, 400, 256, 2
G = B * T

_K_LO = 111999
_K_HI = 112000
_W_LO = 0.703125
_W_HI = 0.296875
_BN_SCALE_EPS = 1e-5
_LN_EPS = 1e-5


def _adj_kernel(a_ref, adj_ref):
    x = a_ref[0]
    ub = jax.lax.bitcast_convert_type(x, jnp.uint32)
    neg = ub >= jnp.uint32(0x80000000)
    u = jnp.where(neg, ~ub, ub | jnp.uint32(0x80000000))

    v = jnp.uint32(0)
    k_lo = jnp.float32(_K_LO)
    for bit in range(31, -1, -1):
        t = v | jnp.uint32(1 << bit)
        cnt = jnp.sum((u < t).astype(jnp.float32))
        v = jnp.where(cnt <= k_lo, t, v)

    bits1 = jnp.where(v >= jnp.uint32(0x80000000), v - jnp.uint32(0x80000000), ~v)
    x1 = jax.lax.bitcast_convert_type(bits1, jnp.float32)

    cnt_le = jnp.sum((u <= v).astype(jnp.float32))
    nxt = jnp.min(jnp.where(x > x1, x, jnp.float32(jnp.inf)))
    x2 = jnp.where(cnt_le >= jnp.float32(_K_HI + 1), x1, nxt)

    thr = x1 * jnp.float32(_W_LO) + x2 * jnp.float32(_W_HI)
    adj_ref[0] = (x > thr).astype(jnp.bfloat16)


def _h0_kernel(x_ref, w_ref, b_ref, o_ref):
    o_ref[:] = (
        jnp.dot(x_ref[:], w_ref[:], preferred_element_type=jnp.float32) + b_ref[:]
    )


def _gin_kernel(adj_ref, h_ref, eps_ref, w1_ref, b1_ref, g1_ref, bb1_ref,
                w2_ref, b2_ref, g2_ref, bb2_ref, we_ref, be_ref, ge_ref,
                bge_ref, wa_ref, ba_ref, ho_ref, hro_ref):
    s = jax.lax.rsqrt(jnp.float32(1.0 + _BN_SCALE_EPS))
    adj = adj_ref[0].astype(jnp.float32)
    h = h_ref[0]
    agg = jnp.dot(adj, h, preferred_element_type=jnp.float32) + eps_ref[0, 0] * h
    z1 = jnp.dot(agg, w1_ref[:], preferred_element_type=jnp.float32) + b1_ref[:]
    h1 = jnp.maximum(z1 * s * g1_ref[:] + bb1_ref[:], 0.0)
    z2 = jnp.dot(h1, w2_ref[:], preferred_element_type=jnp.float32) + b2_ref[:]
    h2 = jnp.maximum(z2 * s * g2_ref[:] + bb2_ref[:], 0.0)
    ho_ref[0] = h2

    xr = jnp.mean(h2, axis=0, keepdims=True)
    e = jnp.dot(xr, we_ref[:], preferred_element_type=jnp.float32) + be_ref[:]
    e = e * s * ge_ref[:] + bge_ref[:]
    emb = 0.5 * e * (1.0 + jax.lax.erf(e * jnp.float32(0.7071067811865476)))
    attn = jax.nn.sigmoid(
        jnp.dot(emb, wa_ref[:], preferred_element_type=jnp.float32) + ba_ref[:]
    )
    hro_ref[0] = jnp.dot(attn, h2, preferred_element_type=jnp.float32) * (1.0 / N)


def _ln(x, g, b):
    m = jnp.mean(x, axis=-1, keepdims=True)
    d = x - m
    v = jnp.mean(d * d, axis=-1, keepdims=True)
    return d / jnp.sqrt(v + jnp.float32(_LN_EPS)) * g + b


def _tf_kernel(x_ref, wqkv_ref, bqkv_ref, wo_ref, bo_ref, ln1g_ref, ln1b_ref,
               wm1_ref, bm1_ref, wm2_ref, bm2_ref, ln2g_ref, ln2b_ref,
               wl_ref, bl_ref, o_ref):
    x = x_ref[:].reshape(B * T, C)
    qkv = jnp.dot(x, wqkv_ref[:], preferred_element_type=jnp.float32) + bqkv_ref[:]
    q = qkv[:, :C].reshape(B, T, C)
    k = qkv[:, C:2 * C].reshape(B, T, C)
    v = qkv[:, 2 * C:].reshape(B, T, C)
    atts = []
    scale = jnp.float32(1.0 / 16.0)
    for b in range(B):
        aw = jnp.dot(q[b], k[b].T, preferred_element_type=jnp.float32) * scale
        aw = aw - jnp.max(aw, axis=-1, keepdims=True)
        ew = jnp.exp(aw)
        aw = ew / jnp.sum(ew, axis=-1, keepdims=True)
        atts.append(jnp.dot(aw, v[b], preferred_element_type=jnp.float32))
    att = jnp.concatenate(atts, axis=0)
    att = jnp.dot(att, wo_ref[:], preferred_element_type=jnp.float32) + bo_ref[:]
    x1 = _ln(att, ln1g_ref[:], ln1b_ref[:])
    z = jnp.maximum(
        jnp.dot(x1, wm1_ref[:], preferred_element_type=jnp.float32) + bm1_ref[:], 0.0
    )
    x2 = jnp.dot(z, wm2_ref[:], preferred_element_type=jnp.float32) + bm2_ref[:]
    xa = _ln(x1 + x2, ln2g_ref[:], ln2b_ref[:])
    lat = jnp.sum(xa.reshape(B, T, C), axis=1)
    o_ref[:] = jnp.dot(lat, wl_ref[:], preferred_element_type=jnp.float32) + bl_ref[:]


def _row(x):
    return x.reshape(1, -1)


@jax.jit
def kernel(fMRI_v, fMRI_a, params):
    a3 = fMRI_a.reshape(G, N, N)
    adj = pl.pallas_call(
        _adj_kernel,
        grid=(G,),
        in_specs=[pl.BlockSpec((1, N, N), lambda i: (i, 0, 0))],
        out_specs=pl.BlockSpec((1, N, N), lambda i: (i, 0, 0)),
        out_shape=jax.ShapeDtypeStruct((G, N, N), jnp.bfloat16),
    )(a3)

    RW = 512
    v2 = fMRI_v.reshape(B * T * N, N)
    h0 = pl.pallas_call(
        _h0_kernel,
        grid=(v2.shape[0] // RW,),
        in_specs=[
            pl.BlockSpec((RW, N), lambda i: (i, 0)),
            pl.BlockSpec((N, C), lambda i: (0, 0)),
            pl.BlockSpec((1, C), lambda i: (0, 0)),
        ],
        out_specs=pl.BlockSpec((RW, C), lambda i: (i, 0)),
        out_shape=jax.ShapeDtypeStruct((v2.shape[0], C), jnp.float32),
    )(v2, params["W0"], _row(params["b0"]))

    h = h0.reshape(G, N, C)
    logit = jnp.zeros((B, NC), jnp.float32)
    const = lambda *_: (0, 0)
    for p in params["layers"]:
        h, hro = pl.pallas_call(
            _gin_kernel,
            grid=(G,),
            in_specs=[
                pl.BlockSpec((1, N, N), lambda i: (i, 0, 0)),
                pl.BlockSpec((1, N, C), lambda i: (i, 0, 0)),
            ] + [pl.BlockSpec(s, const) for s in [
                (1, 1), (C, C), (1, C), (1, C), (1, C),
                (C, C), (1, C), (1, C), (1, C),
                (C, C), (1, C), (1, C), (1, C),
                (C, N), (1, N),
            ]],
            out_specs=[
                pl.BlockSpec((1, N, C), lambda i: (i, 0, 0)),
                pl.BlockSpec((1, 1, C), lambda i: (i, 0, 0)),
            ],
            out_shape=[
                jax.ShapeDtypeStruct((G, N, C), jnp.float32),
                jax.ShapeDtypeStruct((G, 1, C), jnp.float32),
            ],
        )(adj, h, p["eps"], p["W1"], _row(p["b1"]), _row(p["g1"]), _row(p["bb1"]),
          p["W2"], _row(p["b2"]), _row(p["g2"]), _row(p["bb2"]),
          p["We"], _row(p["be"]), _row(p["ge"]), _row(p["bge"]),
          p["Wa"], _row(p["ba"]))

        lo = pl.pallas_call(
            _tf_kernel,
            in_specs=[pl.BlockSpec((B, T, C), lambda: (0, 0, 0))] + [
                pl.BlockSpec(s, lambda: (0, 0)) for s in [
                    (C, 3 * C), (1, 3 * C), (C, C), (1, C),
                    (1, C), (1, C), (C, 2 * C), (1, 2 * C),
                    (2 * C, C), (1, C), (1, C), (1, C),
                    (C, NC), (1, NC),
                ]],
            out_specs=pl.BlockSpec((B, NC), lambda: (0, 0)),
            out_shape=jax.ShapeDtypeStruct((B, NC), jnp.float32),
        )(hro.reshape(B, T, C), p["Wqkv"], _row(p["bqkv"]), p["Wo"], _row(p["bo"]),
          _row(p["ln1g"]), _row(p["ln1b"]), p["Wm1"], _row(p["bm1"]),
          p["Wm2"], _row(p["bm2"]), _row(p["ln2g"]), _row(p["ln2b"]),
          p["Wl"], _row(p["bl"]))
        logit = logit + lo
    return logit


# speedup vs baseline: 31.4270x; 2.5244x over previous
import jax
import jax.numpy as jnp
from jax.experimental import pallas as pl
from jax.experimental.pallas import tpu as pltpu

B, T, N, C, NC = 4, 32, 400, 256, 2
G = B * T

_BPS = 4
_K_LO = 111999
_W_LO = 0.703125
_W_HI = 0.296875
_BN_SCALE_EPS = 1e-5
_LN_EPS = 1e-5


def _exact_threshold(x):
    ub = jax.lax.bitcast_convert_type(x, jnp.uint32)
    neg = ub >= jnp.uint32(0x80000000)
    u = jnp.where(neg, ~ub, ub | jnp.uint32(0x80000000))

    v = jnp.zeros((_BPS, 1, 1), jnp.uint32)
    k_lo = jnp.float32(_K_LO)
    for bit in range(31, -1, -1):
        t = v | jnp.uint32(1 << bit)
        cnt = jnp.sum((u < t).astype(jnp.float32), axis=(1, 2), keepdims=True)
        v = jnp.where(cnt <= k_lo, t, v)

    bits1 = jnp.where(v >= jnp.uint32(0x80000000), v - jnp.uint32(0x80000000), ~v)
    x1 = jax.lax.bitcast_convert_type(bits1, jnp.float32)

    cnt_le = jnp.sum((u <= v).astype(jnp.float32), axis=(1, 2), keepdims=True)
    nxt = jnp.min(jnp.where(x > x1, x, jnp.float32(jnp.inf)), axis=(1, 2),
                  keepdims=True)
    x2 = jnp.where(cnt_le >= jnp.float32(_K_LO + 2), x1, nxt)
    return x1 * jnp.float32(_W_LO) + x2 * jnp.float32(_W_HI)


def _dotd(a, b):
    return jnp.dot(a.astype(jnp.bfloat16), b.astype(jnp.bfloat16),
                   preferred_element_type=jnp.float32)


def _gin_stage(adj, h, p, s):
    (eps, w1, b1, g1, bb1, w2, b2, g2, bb2, we, be, ge, bge, wa, ba) = p
    agg = _dotd(adj, h) + eps * h
    z1 = _dotd(agg, w1) + b1
    h1 = jnp.maximum(z1 * s * g1 + bb1, 0.0)
    z2 = _dotd(h1, w2) + b2
    h2 = jnp.maximum(z2 * s * g2 + bb2, 0.0)

    xr = jnp.mean(h2, axis=0, keepdims=True)
    e = _dotd(xr, we) + be
    e = e * s * ge + bge
    emb = 0.5 * e * (1.0 + jax.lax.erf(e * jnp.float32(0.7071067811865476)))
    attn = jax.nn.sigmoid(
        _dotd(emb, wa) + ba
    )
    hro = jnp.dot(attn, h2, preferred_element_type=jnp.float32, precision=jax.lax.Precision.HIGHEST) * (1.0 / N)
    return h2, hro


def _block_kernel(a_ref, v_ref, w0_ref, b0_ref,
                  eps1, w1a, b1a, g1a, bb1a, w2a, b2a, g2a, bb2a,
                  wea, bea, gea, bgea, waa, baa,
                  eps2, w1b, b1b, g1b, bb1b, w2b, b2b, g2b, bb2b,
                  web, beb, geb, bgeb, wab, bab,
                  ro1_ref, ro2_ref, adj_s, h0_s):
    s = jax.lax.rsqrt(jnp.float32(1.0 + _BN_SCALE_EPS))
    p1 = (eps1[0, 0], w1a[:], b1a[:], g1a[:], bb1a[:], w2a[:], b2a[:],
          g2a[:], bb2a[:], wea[:], bea[:], gea[:], bgea[:], waa[:], baa[:])
    p2 = (eps2[0, 0], w1b[:], b1b[:], g1b[:], bb1b[:], w2b[:], b2b[:],
          g2b[:], bb2b[:], web[:], beb[:], geb[:], bgeb[:], wab[:], bab[:])
    for j in range(_BPS):
        h1, ro1 = _gin_stage(adj_s[j], h0_s[j], p1, s)
        _, ro2 = _gin_stage(adj_s[j], h1, p2, s)
        ro1_ref[j] = ro1
        ro2_ref[j] = ro2

    x = a_ref[:]
    thr = _exact_threshold(x)
    adj_s[:] = (x > thr).astype(jnp.bfloat16)
    for j in range(_BPS):
        h0_s[j] = (_dotd(v_ref[j], w0_ref[:]) + b0_ref[:]).astype(jnp.bfloat16)


def _ln(x, g, b):
    m = jnp.mean(x, axis=-1, keepdims=True)
    d = x - m
    v = jnp.mean(d * d, axis=-1, keepdims=True)
    return d / jnp.sqrt(v + jnp.float32(_LN_EPS)) * g + b


def _tf_stage(x, wqkv, bqkv, wo, bo, ln1g, ln1b, wm1, bm1, wm2, bm2,
              ln2g, ln2b, wl, bl):
    qkv = _dotd(x.reshape(B * T, C), wqkv) + bqkv
    q = qkv[:, :C].reshape(B, T, C)
    k = qkv[:, C:2 * C].reshape(B, T, C)
    v = qkv[:, 2 * C:].reshape(B, T, C)
    atts = []
    scale = jnp.float32(1.0 / 16.0)
    for b in range(B):
        aw = _dotd(q[b], k[b].T) * scale
        aw = aw - jnp.max(aw, axis=-1, keepdims=True)
        ew = jnp.exp(aw)
        aw = ew / jnp.sum(ew, axis=-1, keepdims=True)
        atts.append(_dotd(aw, v[b]))
    att = jnp.concatenate(atts, axis=0)
    att = _dotd(att, wo) + bo
    x1 = _ln(att, ln1g, ln1b)
    z = jnp.maximum(_dotd(x1, wm1) + bm1, 0.0)
    x2 = _dotd(z, wm2) + bm2
    xa = _ln(x1 + x2, ln2g, ln2b)
    lat = jnp.sum(xa.reshape(B, T, C), axis=1)
    return _dotd(lat, wl) + bl


def _tf_kernel(x1_ref, x2_ref, *refs):
    pa = [r[:] for r in refs[:14]]
    pb = [r[:] for r in refs[14:28]]
    o_ref = refs[28]
    o_ref[:] = (_tf_stage(x1_ref[:], *pa) + _tf_stage(x2_ref[:], *pb))


def _row(x):
    return x.reshape(1, -1)


def _layer_args(p):
    return (p["eps"], p["W1"], _row(p["b1"]), _row(p["g1"]), _row(p["bb1"]),
            p["W2"], _row(p["b2"]), _row(p["g2"]), _row(p["bb2"]),
            p["We"], _row(p["be"]), _row(p["ge"]), _row(p["bge"]),
            p["Wa"], _row(p["ba"]))


def _tf_args(p):
    return (p["Wqkv"], _row(p["bqkv"]), p["Wo"], _row(p["bo"]),
            _row(p["ln1g"]), _row(p["ln1b"]), p["Wm1"], _row(p["bm1"]),
            p["Wm2"], _row(p["bm2"]), _row(p["ln2g"]), _row(p["ln2b"]),
            p["Wl"], _row(p["bl"]))


@jax.jit
def kernel(fMRI_v, fMRI_a, params):
    a3 = fMRI_a.reshape(G, N, N)
    v3 = fMRI_v.reshape(G, N, N)
    pa, pb = params["layers"][0], params["layers"][1]

    nsteps = G // _BPS + 1
    clamp_in = lambda i: (jnp.minimum(i, G // _BPS - 1), 0, 0)
    const2 = lambda i: (0, 0)
    prev = lambda i: (jnp.maximum(i - 1, 0), 0, 0)
    layer_specs = [pl.BlockSpec(s, const2) for s in [
        (1, 1), (C, C), (1, C), (1, C), (1, C),
        (C, C), (1, C), (1, C), (1, C),
        (C, C), (1, C), (1, C), (1, C),
        (C, N), (1, N),
    ]]

    ro1, ro2 = pl.pallas_call(
        _block_kernel,
        grid=(nsteps,),
        in_specs=[
            pl.BlockSpec((_BPS, N, N), clamp_in),
            pl.BlockSpec((_BPS, N, N), clamp_in),
            pl.BlockSpec((N, C), const2),
            pl.BlockSpec((1, C), const2),
        ] + layer_specs + layer_specs,
        out_specs=[
            pl.BlockSpec((_BPS, 1, C), prev),
            pl.BlockSpec((_BPS, 1, C), prev),
        ],
        out_shape=[
            jax.ShapeDtypeStruct((G, 1, C), jnp.float32),
            jax.ShapeDtypeStruct((G, 1, C), jnp.float32),
        ],
        scratch_shapes=[
            pltpu.VMEM((_BPS, N, N), jnp.bfloat16),
            pltpu.VMEM((_BPS, N, C), jnp.bfloat16),
        ],
    )(a3, v3, params["W0"], _row(params["b0"]),
      *_layer_args(pa), *_layer_args(pb))

    tf_specs = [pl.BlockSpec(s, lambda i: (0, 0)) for s in [
        (C, 3 * C), (1, 3 * C), (C, C), (1, C),
        (1, C), (1, C), (C, 2 * C), (1, 2 * C),
        (2 * C, C), (1, C), (1, C), (1, C),
        (C, NC), (1, NC),
    ]]
    logit = pl.pallas_call(
        _tf_kernel,
        grid=(1,),
        in_specs=[
            pl.BlockSpec((B, T, C), lambda i: (0, 0, 0)),
            pl.BlockSpec((B, T, C), lambda i: (0, 0, 0)),
        ] + tf_specs + tf_specs,
        out_specs=pl.BlockSpec((B, NC), lambda i: (0, 0)),
        out_shape=jax.ShapeDtypeStruct((B, NC), jnp.float32),
    )(ro1.reshape(B, T, C), ro2.reshape(B, T, C),
      *_tf_args(pa), *_tf_args(pb))
    return logit
